# baseline (speedup 1.0000x reference)
import functools

import jax
import jax.numpy as jnp
import numpy as np
from jax import lax
from jax.experimental import pallas as pl
from jax.experimental.pallas import tpu as pltpu
from jax.experimental.pallas import tpu_sc as plsc

_STRIPE = 4
_DIRECTIONS = ("h_fwd", "h_bwd", "v_fwd", "v_bwd")


def _nss_indices(H, W, stripe_width, direction):
    L = H * W
    indices = np.zeros(L, dtype=np.int64)
    if direction.startswith("h"):
        pos = 0
        num_stripes = (H + stripe_width - 1) // stripe_width
        for s in range(num_stripes):
            row_start = s * stripe_width
            row_end = min(row_start + stripe_width, H)
            for local_r, r in enumerate(range(row_start, row_end)):
                if local_r % 2 == 0:
                    for c in range(W):
                        indices[pos] = r * W + c
                        pos += 1
                else:
                    for c in range(W - 1, -1, -1):
                        indices[pos] = r * W + c
                        pos += 1
        if direction == "h_bwd":
            indices = indices[::-1].copy()
    else:
        pos = 0
        num_stripes = (W + stripe_width - 1) // stripe_width
        for s in range(num_stripes):
            col_start = s * stripe_width
            col_end = min(col_start + stripe_width, W)
            for local_c, c in enumerate(range(col_start, col_end)):
                if local_c % 2 == 0:
                    for r in range(H):
                        indices[pos] = r * W + c
                        pos += 1
                else:
                    for r in range(H - 1, -1, -1):
                        indices[pos] = r * W + c
                        pos += 1
        if direction == "v_bwd":
            indices = indices[::-1].copy()
    return indices


@functools.lru_cache(maxsize=None)
def _scatter_tables(N, H, W, b_ch):
    L = H * W
    NL = N * L
    n_ch = L // b_ch
    inv = []
    for d in _DIRECTIONS:
        idx = _nss_indices(H, W, _STRIPE, d)
        inv_d = np.argsort(idx)
        inv.append(inv_d)
    tab = np.empty((N, n_ch, len(_DIRECTIONS), b_ch), dtype=np.int32)
    for n in range(N):
        for i in range(n_ch):
            q = np.arange(i * b_ch, (i + 1) * b_ch)
            for d in range(len(_DIRECTIONS)):
                tab[n, i, d] = d * NL + n * L + inv[d][q]
    return tab.reshape(N * n_ch * len(_DIRECTIONS), b_ch)


@functools.lru_cache(maxsize=None)
def _make_sc_scatter(N, L, D, b_ch):
    NL = N * L
    B = 4 * NL
    ND = len(_DIRECTIONS)
    n_ch = L // b_ch
    info = plsc.get_sparse_core_info()
    NC, NS = info.num_cores, info.num_subcores
    NW = NC * NS
    assert N == NW and L % b_ch == 0
    mesh = plsc.VectorSubcoreMesh(core_axis_name="c", subcore_axis_name="s")

    @functools.partial(
        pl.kernel,
        mesh=mesh,
        out_type=jax.ShapeDtypeStruct((B, D), jnp.float32),
        scratch_types=[
            pltpu.VMEM((n_ch * ND, b_ch), jnp.int32),
            pltpu.VMEM((b_ch, D), jnp.float32),
            pltpu.VMEM((b_ch, D), jnp.float32),
            pltpu.VMEM_SHARED((NS, 2, b_ch, D), jnp.float32),
            pltpu.SemaphoreType.DMA,
            pltpu.SemaphoreType.DMA,
            pltpu.SemaphoreType.DMA,
            pltpu.SemaphoreType.DMA,
            pltpu.SemaphoreType.DMA,
            pltpu.SemaphoreType.DMA,
        ],
    )
    def scatter_kernel(table_hbm, sdx_hbm, out_hbm, sdx_all,
                       rows0, rows1, spmem, ss0, ss1, sr0, sr1, sw0, sw1):
        t = lax.axis_index("s") * NC + lax.axis_index("c")
        sid = lax.axis_index("s")
        rows = (rows0, rows1)
        sem_s = (ss0, ss1)
        sem_r = (sr0, sr1)
        sem_w = (sw0, sw1)

        pltpu.sync_copy(sdx_hbm.at[pl.ds(t * n_ch * ND, n_ch * ND)], sdx_all)

        def stage_chunk(i):
            bi = i % 2
            return pltpu.async_copy(
                table_hbm.at[pl.ds(t * L + i * b_ch, b_ch)],
                spmem.at[sid, bi], sem_s[bi])

        def write_chunk(j):
            bj = j % 2
            return [
                pltpu.async_copy(
                    rows[bj], out_hbm.at[sdx_all.at[j * ND + d]], sem_w[bj])
                for d in range(ND)
            ]

        stage_h = [None] * n_ch
        write_h = [None] * n_ch
        stage_h[0] = stage_chunk(0)
        for i in range(n_ch):
            b = i % 2
            if i + 1 < n_ch:
                stage_h[i + 1] = stage_chunk(i + 1)
            stage_h[i].wait()
            if i >= 2:
                for h in write_h[i - 2]:
                    h.wait()
            pltpu.async_copy(spmem.at[sid, b], rows[b], sem_r[b]).wait()
            write_h[i] = write_chunk(i)
        for j in (n_ch - 2, n_ch - 1):
            for h in write_h[j]:
                h.wait()

    return scatter_kernel


def kernel(x_2d):
    N, H, W, C = x_2d.shape
    L = H * W
    b_ch = 64
    table = x_2d.reshape(N * L, C)
    sdx = jnp.asarray(_scatter_tables(N, H, W, b_ch))
    out = _make_sc_scatter(N, L, C, b_ch)(table, sdx)
    return out.reshape(4 * N, L, C)


# speedup vs baseline: 1.0739x; 1.0739x over previous
import functools

import jax
import jax.numpy as jnp
import numpy as np
from jax import lax
from jax.experimental import pallas as pl
from jax.experimental.pallas import tpu as pltpu
from jax.experimental.pallas import tpu_sc as plsc

_STRIPE = 4
_DIRECTIONS = ("h_fwd", "h_bwd", "v_fwd", "v_bwd")


def _nss_indices(H, W, stripe_width, direction):
    L = H * W
    indices = np.zeros(L, dtype=np.int64)
    if direction.startswith("h"):
        pos = 0
        num_stripes = (H + stripe_width - 1) // stripe_width
        for s in range(num_stripes):
            row_start = s * stripe_width
            row_end = min(row_start + stripe_width, H)
            for local_r, r in enumerate(range(row_start, row_end)):
                if local_r % 2 == 0:
                    for c in range(W):
                        indices[pos] = r * W + c
                        pos += 1
                else:
                    for c in range(W - 1, -1, -1):
                        indices[pos] = r * W + c
                        pos += 1
        if direction == "h_bwd":
            indices = indices[::-1].copy()
    else:
        pos = 0
        num_stripes = (W + stripe_width - 1) // stripe_width
        for s in range(num_stripes):
            col_start = s * stripe_width
            col_end = min(col_start + stripe_width, W)
            for local_c, c in enumerate(range(col_start, col_end)):
                if local_c % 2 == 0:
                    for r in range(H):
                        indices[pos] = r * W + c
                        pos += 1
                else:
                    for r in range(H - 1, -1, -1):
                        indices[pos] = r * W + c
                        pos += 1
        if direction == "v_bwd":
            indices = indices[::-1].copy()
    return indices


@functools.lru_cache(maxsize=None)
def _scatter_tables(N, H, W, b_ch):
    L = H * W
    NL = N * L
    n_ch = L // b_ch
    inv = []
    for d in _DIRECTIONS:
        idx = _nss_indices(H, W, _STRIPE, d)
        inv_d = np.argsort(idx)
        inv.append(inv_d)
    tab = np.empty((N, n_ch, len(_DIRECTIONS), b_ch), dtype=np.int32)
    for n in range(N):
        for i in range(n_ch):
            q = np.arange(i * b_ch, (i + 1) * b_ch)
            for d in range(len(_DIRECTIONS)):
                tab[n, i, d] = d * NL + n * L + inv[d][q]
    return tab.reshape(N * n_ch * len(_DIRECTIONS), b_ch)


@functools.lru_cache(maxsize=None)
def _make_sc_scatter(N, L, D, b_ch):
    NL = N * L
    B = 4 * NL
    ND = len(_DIRECTIONS)
    n_ch = L // b_ch
    info = plsc.get_sparse_core_info()
    NC, NS = info.num_cores, info.num_subcores
    NW = NC * NS
    assert N == NW and L % b_ch == 0
    mesh = plsc.VectorSubcoreMesh(core_axis_name="c", subcore_axis_name="s")

    @functools.partial(
        pl.kernel,
        mesh=mesh,
        out_type=jax.ShapeDtypeStruct((B, D), jnp.float32),
        scratch_types=[
            pltpu.VMEM((n_ch * ND, b_ch), jnp.int32),
            pltpu.VMEM((b_ch, D), jnp.float32),
            pltpu.VMEM((b_ch, D), jnp.float32),
            pltpu.SemaphoreType.DMA,
            pltpu.SemaphoreType.DMA,
            pltpu.SemaphoreType.DMA,
            pltpu.SemaphoreType.DMA,
        ],
    )
    def scatter_kernel(table_hbm, sdx_hbm, out_hbm, sdx_all,
                       rows0, rows1, sr0, sr1, sw0, sw1):
        t = lax.axis_index("s") * NC + lax.axis_index("c")
        rows = (rows0, rows1)
        sem_r = (sr0, sr1)
        sem_w = (sw0, sw1)

        pltpu.sync_copy(sdx_hbm.at[pl.ds(t * n_ch * ND, n_ch * ND)], sdx_all)

        def write_chunk(j):
            bj = j % 2
            return [
                pltpu.async_copy(
                    rows[bj], out_hbm.at[sdx_all.at[j * ND + d]], sem_w[bj])
                for d in range(ND)
            ]

        read_h = [None] * n_ch
        write_h = [None] * n_ch
        for i in range(n_ch):
            b = i % 2
            if i >= 2:
                for h in write_h[i - 2]:
                    h.wait()
            read_h[i] = pltpu.async_copy(
                table_hbm.at[pl.ds(t * L + i * b_ch, b_ch)], rows[b],
                sem_r[b])
            if i >= 1:
                read_h[i - 1].wait()
                write_h[i - 1] = write_chunk(i - 1)
        read_h[n_ch - 1].wait()
        write_h[n_ch - 1] = write_chunk(n_ch - 1)
        for j in (n_ch - 2, n_ch - 1):
            for h in write_h[j]:
                h.wait()

    return scatter_kernel


def kernel(x_2d):
    N, H, W, C = x_2d.shape
    L = H * W
    b_ch = 128
    table = x_2d.reshape(N * L, C)
    sdx = jnp.asarray(_scatter_tables(N, H, W, b_ch))
    out = _make_sc_scatter(N, L, C, b_ch)(table, sdx)
    return out.reshape(4 * N, L, C)
